# baseline (speedup 1.0000x reference)
import functools

import jax
import jax.numpy as jnp
from jax import lax
from jax.experimental import pallas as pl
from jax.experimental.pallas import tpu as pltpu
from jax.experimental.pallas import tpu_sc as plsc

LANES = 16


@functools.lru_cache(maxsize=None)
def _make_lookup(S, T, V, D):
    info = plsc.get_sparse_core_info()
    NC, NS = info.num_cores, info.num_subcores
    NW = NC * NS
    assert S % NW == 0
    s_per_w = S // NW
    assert s_per_w % 2 == 0
    TP = (T + LANES - 1) // LANES * LANES

    mesh = plsc.VectorSubcoreMesh(core_axis_name="c", subcore_axis_name="s")

    @functools.partial(
        pl.kernel,
        mesh=mesh,
        out_type=jax.ShapeDtypeStruct((S, T, D), jnp.float32),
        scratch_types=[
            pltpu.VMEM((V, D), jnp.float32),
            pltpu.VMEM((s_per_w, TP), jnp.int32),
            pltpu.VMEM((T, D), jnp.float32),
            pltpu.VMEM((T, D), jnp.float32),
            pltpu.SemaphoreType.DMA,
            pltpu.SemaphoreType.DMA,
        ],
        compiler_params=pltpu.CompilerParams(use_tc_tiling_on_sc=True),
    )
    def lookup(idx_hbm, table_hbm, out_hbm, table_v, idx_v, buf0, buf1,
               sem0, sem1):
        bufs = (buf0, buf1)
        sems = (sem0, sem1)

        wid = lax.axis_index("s") * NC + lax.axis_index("c")
        base = wid * s_per_w
        pltpu.sync_copy(table_hbm, table_v)
        pltpu.sync_copy(idx_hbm.at[wid], idx_v)

        def expand_rows(g, buf, k, nrows):
            iv = idx_v[g, pl.ds(k * LANES, LANES)]
            for l in range(nrows):
                i = iv[l]
                r = k * LANES + l
                for c in range(D // LANES):
                    sl = pl.ds(c * LANES, LANES)
                    buf[r, sl] = table_v[i, sl]

        def compute(g, buf):
            def block(k, carry):
                expand_rows(g, buf, k, LANES)
                return carry

            lax.fori_loop(0, T // LANES, block, 0)
            if T % LANES:
                expand_rows(g, buf, T // LANES, T % LANES)

        def start_write(g, b):
            pltpu.async_copy(bufs[b], out_hbm.at[base + g], sems[b])

        def wait_write(g, b):
            pltpu.make_async_copy(bufs[b], out_hbm.at[base + g], sems[b]).wait()

        for g in (0, 1):
            compute(g, bufs[g])
            start_write(g, g)

        def pair(q, carry):
            for j in range(2):
                g = 2 * q + j
                wait_write(g - 2, j)
                compute(g, bufs[j])
                start_write(g, j)
            return carry

        lax.fori_loop(1, s_per_w // 2, pair, 0)

        wait_write(s_per_w - 2, 0)
        wait_write(s_per_w - 1, 1)

    return lookup


def kernel(input_ids, embed_weight):
    S, T = input_ids.shape
    V, D = embed_weight.shape
    info = plsc.get_sparse_core_info()
    NW = info.num_cores * info.num_subcores
    idx = input_ids.reshape(NW, S // NW, T).astype(jnp.int32)
    TP = (T + LANES - 1) // LANES * LANES
    if TP != T:
        idx = jnp.pad(idx, ((0, 0), (0, 0), (0, TP - T)))
    return _make_lookup(S, T, V, D)(idx, embed_weight)
